# baseline (speedup 1.0000x reference)
import functools

import jax
import jax.numpy as jnp
from jax import lax
from jax.experimental import pallas as pl
from jax.experimental.pallas import tpu as pltpu
from jax.experimental.pallas import tpu_sc as plsc

_NW = 32
_CHUNK = 800


@functools.cache
def _build(n_flat: int, d: int):
    assert n_flat % (_NW * _CHUNK) == 0
    bpw = n_flat // _NW
    nchunk = bpw // _CHUNK

    mesh = plsc.VectorSubcoreMesh(core_axis_name="c", subcore_axis_name="s")

    @functools.partial(
        pl.kernel,
        out_type=jax.ShapeDtypeStruct((n_flat, d), jnp.float32),
        mesh=mesh,
        scratch_types=[
            pltpu.VMEM((_CHUNK,), jnp.int32),
            pltpu.VMEM((_CHUNK, d), jnp.float32),
            pltpu.SemaphoreType.DMA,
        ],
        compiler_params=pltpu.CompilerParams(use_tc_tiling_on_sc=False),
    )
    def gather_kernel(idx_hbm, table_hbm, out_hbm, idx_v, rows_v, sem):
        wid = lax.axis_index("s") * 2 + lax.axis_index("c")
        base = wid * bpw

        def chunk_body(i, carry):
            off = base + i * _CHUNK
            pltpu.sync_copy(idx_hbm.at[pl.ds(off, _CHUNK)], idx_v)
            pltpu.async_copy(table_hbm.at[idx_v], rows_v, sem).wait()
            pltpu.sync_copy(rows_v, out_hbm.at[pl.ds(off, _CHUNK)])
            return carry

        lax.fori_loop(0, nchunk, chunk_body, 0)

    return gather_kernel


def kernel(indices, weight):
    shape = indices.shape
    d = weight.shape[1]
    flat = indices.reshape(-1).astype(jnp.int32)
    out = _build(flat.shape[0], d)(flat, weight)
    return out.reshape(*shape, d)


# speedup vs baseline: 1.0188x; 1.0188x over previous
import functools

import jax
import jax.numpy as jnp
from jax import lax
from jax.experimental import pallas as pl
from jax.experimental.pallas import tpu as pltpu
from jax.experimental.pallas import tpu_sc as plsc

_NW = 32
_CHUNK = 800


@functools.cache
def _build(n_flat: int, d: int):
    assert n_flat % (_NW * _CHUNK) == 0
    bpw = n_flat // _NW
    nchunk = bpw // _CHUNK
    assert nchunk >= 4 and nchunk % 2 == 0

    mesh = plsc.VectorSubcoreMesh(core_axis_name="c", subcore_axis_name="s")

    @functools.partial(
        pl.kernel,
        out_type=jax.ShapeDtypeStruct((n_flat, d), jnp.float32),
        mesh=mesh,
        scratch_types=[
            pltpu.VMEM((2, _CHUNK), jnp.int32),
            pltpu.VMEM((2, _CHUNK, d), jnp.float32),
            pltpu.SemaphoreType.DMA,
            pltpu.SemaphoreType.DMA,
        ],
        compiler_params=pltpu.CompilerParams(use_tc_tiling_on_sc=False),
    )
    def gather_kernel(idx_hbm, table_hbm, out_hbm, idx_v, rows_v, gsem, wsem):
        wid = lax.axis_index("s") * 2 + lax.axis_index("c")
        base = wid * bpw

        def fire_gather(i, b):
            off = base + i * _CHUNK
            pltpu.sync_copy(idx_hbm.at[pl.ds(off, _CHUNK)], idx_v.at[b])
            pltpu.async_copy(table_hbm.at[idx_v.at[b]], rows_v.at[b], gsem)

        def wait_gather(b):
            pltpu.make_async_copy(
                table_hbm.at[pl.ds(0, _CHUNK)], rows_v.at[b], gsem).wait()

        def fire_write(i, b):
            off = base + i * _CHUNK
            pltpu.async_copy(rows_v.at[b], out_hbm.at[pl.ds(off, _CHUNK)], wsem)

        def wait_write(b):
            pltpu.make_async_copy(
                rows_v.at[b], out_hbm.at[pl.ds(0, _CHUNK)], wsem).wait()

        fire_gather(0, 0)
        fire_gather(1, 1)
        wait_gather(0)
        fire_write(0, 0)

        def pair_body(k, carry):
            i0 = 1 + 2 * k
            for p in range(2):
                i = i0 + p
                b = (1 + p) % 2
                nb = 1 - b
                wait_write(nb)
                fire_gather(i + 1, nb)
                wait_gather(b)
                fire_write(i, b)
            return carry

        lax.fori_loop(0, (nchunk - 2) // 2, pair_body, 0)

        b_last = (nchunk - 1) % 2
        wait_gather(b_last)
        fire_write(nchunk - 1, b_last)
        wait_write(0)
        wait_write(1)

    return gather_kernel


def kernel(indices, weight):
    shape = indices.shape
    d = weight.shape[1]
    flat = indices.reshape(-1).astype(jnp.int32)
    out = _build(flat.shape[0], d)(flat, weight)
    return out.reshape(*shape, d)
